# baseline (speedup 1.0000x reference)
import functools

import jax
import jax.numpy as jnp
from jax import lax
from jax.experimental import pallas as pl
from jax.experimental.pallas import tpu as pltpu
from jax.experimental.pallas import tpu_sc as plsc

_INFO = plsc.get_sparse_core_info()
_NC, _NS = _INFO.num_cores, _INFO.num_subcores
_NW = _NC * _NS


def _transpose_call_tc(tt):
    E, V = tt.shape
    W = 20480
    nb = -(-V // (2 * W))
    H = W * nb
    last_blk = (V + W - 1) // W - 1

    def body(a_ref, b_ref, out_ref):
        out_ref[:, 0:E] = a_ref[...].T
        out_ref[:, E:2 * E] = b_ref[...].T

    s = pl.pallas_call(
        body,
        grid=(nb,),
        in_specs=[pl.BlockSpec((E, W), lambda c: (0, c)),
                  pl.BlockSpec((E, W),
                               lambda c: (0, jnp.minimum(c + nb, last_blk)))],
        out_specs=pl.BlockSpec((W, 2 * E), lambda c: (c, 0)),
        out_shape=jax.ShapeDtypeStruct((H, 2 * E), jnp.float32),
    )(tt, tt)
    return s, H


def _gather_call(idx_flat, table_rm, N, D):
    n_per_w = N // _NW
    CH = 800
    NB = 2
    n_ch = n_per_w // CH

    mesh = plsc.VectorSubcoreMesh(core_axis_name="c", subcore_axis_name="s")

    @functools.partial(
        pl.kernel,
        mesh=mesh,
        out_type=jax.ShapeDtypeStruct((N, D), jnp.float32),
        scratch_types=[
            pltpu.VMEM((n_per_w,), jnp.int32),
            [pltpu.VMEM((CH, D), jnp.float32) for _ in range(NB)],
            [pltpu.SemaphoreType.DMA for _ in range(NB)],
            [pltpu.SemaphoreType.DMA for _ in range(NB)],
        ],
        compiler_params=pltpu.CompilerParams(use_tc_tiling_on_sc=False),
    )
    def gather_kernel(idx_hbm, table_hbm, out_hbm, idx_v, bufs, gsems, ssems):
        wid = lax.axis_index("s") * _NC + lax.axis_index("c")
        base = wid * n_per_w

        pltpu.sync_copy(idx_hbm.at[pl.ds(base, n_per_w)], idx_v)

        def start_g(i):
            return pltpu.async_copy(
                table_hbm.at[idx_v.at[pl.ds(i * CH, CH)]], bufs[i % NB],
                gsems[i % NB])

        def start_s(i):
            return pltpu.async_copy(
                bufs[i % NB], out_hbm.at[pl.ds(base + i * CH, CH)],
                ssems[i % NB])

        gcopies = [None] * n_ch
        scopies = [None] * n_ch
        for i in range(min(NB, n_ch)):
            gcopies[i] = start_g(i)
        for i in range(n_ch):
            gcopies[i].wait()
            scopies[i] = start_s(i)
            if i + NB < n_ch:
                scopies[i].wait()
                gcopies[i + NB] = start_g(i + NB)
        for i in range(max(0, n_ch - NB), n_ch):
            scopies[i].wait()

    return gather_kernel(idx_flat, table_rm)


def kernel(x, table):
    B, L = x.shape
    V, D = table.shape
    N = B * L

    tt = jnp.swapaxes(table, 0, 1)
    s, H = _transpose_call_tc(tt)
    table_rm = jnp.reshape(s, (2 * H, D))
    xf = x.reshape(N)
    idx_flat = 2 * (xf % H) + xf // H

    out = _gather_call(idx_flat, table_rm, N, D)
    return out.reshape(B, L, D)


# speedup vs baseline: 1.0081x; 1.0081x over previous
import functools

import jax
import jax.numpy as jnp
from jax import lax
from jax.experimental import pallas as pl
from jax.experimental.pallas import tpu as pltpu
from jax.experimental.pallas import tpu_sc as plsc

_INFO = plsc.get_sparse_core_info()
_NC, _NS = _INFO.num_cores, _INFO.num_subcores
_NW = _NC * _NS


def _transpose_call_tc(tt):
    E, V = tt.shape
    W = 16384
    nb = -(-V // (2 * W))
    H = W * nb
    last_blk = (V + W - 1) // W - 1

    def body(a_ref, b_ref, out_ref):
        out_ref[:, 0:E] = a_ref[...].T
        out_ref[:, E:2 * E] = b_ref[...].T

    s = pl.pallas_call(
        body,
        grid=(nb,),
        in_specs=[pl.BlockSpec((E, W), lambda c: (0, c)),
                  pl.BlockSpec((E, W),
                               lambda c: (0, jnp.minimum(c + nb, last_blk)))],
        out_specs=pl.BlockSpec((W, 2 * E), lambda c: (c, 0)),
        out_shape=jax.ShapeDtypeStruct((H, 2 * E), jnp.float32),
    )(tt, tt)
    return s, H


def _gather_call(idx_flat, table_rm, N, D):
    n_per_w = N // _NW
    CH = 800
    NB = 2
    n_ch = n_per_w // CH

    mesh = plsc.VectorSubcoreMesh(core_axis_name="c", subcore_axis_name="s")

    @functools.partial(
        pl.kernel,
        mesh=mesh,
        out_type=jax.ShapeDtypeStruct((N, D), jnp.float32),
        scratch_types=[
            pltpu.VMEM((n_per_w,), jnp.int32),
            [pltpu.VMEM((CH, D), jnp.float32) for _ in range(NB)],
            [pltpu.SemaphoreType.DMA for _ in range(NB)],
            [pltpu.SemaphoreType.DMA for _ in range(NB)],
        ],
        compiler_params=pltpu.CompilerParams(use_tc_tiling_on_sc=False),
    )
    def gather_kernel(idx_hbm, table_hbm, out_hbm, idx_v, bufs, gsems, ssems):
        wid = lax.axis_index("s") * _NC + lax.axis_index("c")
        base = wid * n_per_w

        pltpu.sync_copy(idx_hbm.at[pl.ds(base, n_per_w)], idx_v)

        def start_g(i):
            return pltpu.async_copy(
                table_hbm.at[idx_v.at[pl.ds(i * CH, CH)]], bufs[i % NB],
                gsems[i % NB])

        def start_s(i):
            return pltpu.async_copy(
                bufs[i % NB], out_hbm.at[pl.ds(base + i * CH, CH)],
                ssems[i % NB])

        gcopies = [None] * n_ch
        scopies = [None] * n_ch
        for i in range(min(NB, n_ch)):
            gcopies[i] = start_g(i)
        for i in range(n_ch):
            gcopies[i].wait()
            scopies[i] = start_s(i)
            if i + NB < n_ch:
                scopies[i].wait()
                gcopies[i + NB] = start_g(i + NB)
        for i in range(max(0, n_ch - NB), n_ch):
            scopies[i].wait()

    return gather_kernel(idx_flat, table_rm)


def kernel(x, table):
    B, L = x.shape
    V, D = table.shape
    N = B * L

    tt = jnp.swapaxes(table, 0, 1)
    s, H = _transpose_call_tc(tt)
    table_rm = jnp.reshape(s, (2 * H, D))
    xf = x.reshape(N)
    idx_flat = 2 * (xf % H) + xf // H

    out = _gather_call(idx_flat, table_rm, N, D)
    return out.reshape(B, L, D)
